# baseline (speedup 1.0000x reference)
import functools

import jax
import jax.numpy as jnp
from jax import lax
from jax.experimental import pallas as pl
from jax.experimental.pallas import tpu as pltpu
from jax.experimental.pallas import tpu_sc as plsc

N = 10000
NPAD = 10240
ATOM_NF = 32
IN_SIZE = 64
LIG_K = 8
REC_K = 6
TW = 256
EG_LL = NPAD * LIG_K
E_RL = NPAD * REC_K
EG_RL = 65536
CH = 128
NW = 32
PER_W_LL = EG_LL // NW
PER_W_RL = EG_RL // NW

_f32 = jnp.float32
_BF = jnp.bfloat16


def _r(x):
    return x.astype(_BF).astype(jnp.float32)


def _silu(x):
    return x * jax.nn.sigmoid(x)


def _pad_rows(a, npad):
    return jnp.pad(a, ((0, npad - a.shape[0]),) + ((0, 0),) * (a.ndim - 1))


def _enc_body(x_ref, w1_ref, b1_ref, w2_ref, b2_ref, o_ref):
    x = x_ref[0]
    h = _silu(jnp.dot(x, w1_ref[0], preferred_element_type=_f32) + b1_ref[0])
    o_ref[0] = jnp.dot(h, w2_ref[0], preferred_element_type=_f32) + b2_ref[0]


def _encode_both(feat2, w1, b1, w2, b2):
    blk = 2048
    return pl.pallas_call(
        _enc_body,
        grid=(2, NPAD // blk),
        in_specs=[
            pl.BlockSpec((1, blk, ATOM_NF), lambda s, i: (s, i, 0)),
            pl.BlockSpec((1, ATOM_NF, 64), lambda s, i: (s, 0, 0)),
            pl.BlockSpec((1, 1, 64), lambda s, i: (s, 0, 0)),
            pl.BlockSpec((1, 64, 64), lambda s, i: (s, 0, 0)),
            pl.BlockSpec((1, 1, 64), lambda s, i: (s, 0, 0)),
        ],
        out_specs=pl.BlockSpec((1, blk, 64), lambda s, i: (s, i, 0)),
        out_shape=jax.ShapeDtypeStruct((2, NPAD, 64), _f32),
    )(feat2, w1, b1, w2, b2)


def _knn_body(k, dst_ref, srcp_ref, o_ref):
    d = dst_ref[...]
    s0 = srcp_ref[0:1, :]
    s1 = srcp_ref[1:2, :]
    s2 = srcp_ref[2:3, :]
    ssq = srcp_ref[3:4, :]
    dd = jnp.sum(d[:, 0:3] * d[:, 0:3], axis=1, keepdims=True)
    dot = (_r(d[:, 0:1]) * _r(s0) + _r(d[:, 1:2]) * _r(s1) + _r(d[:, 2:3]) * _r(s2))
    acc = (dd + ssq) - 2.0 * dot
    iota = lax.broadcasted_iota(jnp.int32, acc.shape, 1)
    for j in range(k):
        m = jnp.min(acc, axis=1, keepdims=True)
        eq = acc == m
        idx = jnp.min(jnp.where(eq, iota, jnp.int32(2**30)), axis=1, keepdims=True)
        o_ref[:, j:j + 1] = idx
        acc = jnp.where(iota == idx, jnp.float32(jnp.inf), acc)


def _knn(dst_pos, srcp, k):
    rk = 256
    return pl.pallas_call(
        functools.partial(_knn_body, k),
        grid=(NPAD // rk,),
        in_specs=[
            pl.BlockSpec((rk, 4), lambda i: (i, 0)),
            pl.BlockSpec((8, NPAD), lambda i: (0, 0)),
        ],
        out_specs=pl.BlockSpec((rk, k), lambda i: (i, 0)),
        out_shape=jax.ShapeDtypeStruct((NPAD, k), jnp.int32),
    )(dst_pos, srcp)


def _proj_body(hl_ref, hr_ref, xl_ref, xr_ref, wt_ll_ref, wt_rl_ref,
               wd_ll_ref, bd_ll_ref, wd_rl_ref, bd_rl_ref,
               tll_ref, trl_ref, dll_ref, drl_ref):
    hl = hl_ref[...]
    hr = hr_ref[...]
    tll_ref[:, 0:128] = jnp.dot(hl, wt_ll_ref[...], preferred_element_type=_f32)
    tll_ref[:, 128:132] = xl_ref[...]
    tll_ref[:, 132:256] = jnp.zeros_like(tll_ref[:, 132:256])
    trl_ref[:, 0:128] = jnp.dot(hr, wt_rl_ref[...], preferred_element_type=_f32)
    trl_ref[:, 128:132] = xr_ref[...]
    trl_ref[:, 132:256] = jnp.zeros_like(trl_ref[:, 132:256])
    dll_ref[...] = jnp.dot(hl, wd_ll_ref[...], preferred_element_type=_f32) + bd_ll_ref[...]
    drl_ref[...] = jnp.dot(hl, wd_rl_ref[...], preferred_element_type=_f32) + bd_rl_ref[...]


def _project(h_lig, h_rec, x_lig, x_rec, wt_ll, wt_rl, wd_ll, bd_ll, wd_rl, bd_rl):
    bp = 1024
    full = lambda i: (0, 0)
    return pl.pallas_call(
        _proj_body,
        grid=(NPAD // bp,),
        in_specs=[
            pl.BlockSpec((bp, 64), lambda i: (i, 0)),
            pl.BlockSpec((bp, 64), lambda i: (i, 0)),
            pl.BlockSpec((bp, 4), lambda i: (i, 0)),
            pl.BlockSpec((bp, 4), lambda i: (i, 0)),
            pl.BlockSpec((64, 128), full),
            pl.BlockSpec((64, 128), full),
            pl.BlockSpec((64, 128), full),
            pl.BlockSpec((1, 128), full),
            pl.BlockSpec((64, 128), full),
            pl.BlockSpec((1, 128), full),
        ],
        out_specs=[
            pl.BlockSpec((bp, TW), lambda i: (i, 0)),
            pl.BlockSpec((bp, TW), lambda i: (i, 0)),
            pl.BlockSpec((bp, 128), lambda i: (i, 0)),
            pl.BlockSpec((bp, 128), lambda i: (i, 0)),
        ],
        out_shape=[
            jax.ShapeDtypeStruct((NPAD, TW), _f32),
            jax.ShapeDtypeStruct((NPAD, TW), _f32),
            jax.ShapeDtypeStruct((NPAD, 128), _f32),
            jax.ShapeDtypeStruct((NPAD, 128), _f32),
        ],
    )(h_lig, h_rec, x_lig, x_rec, wt_ll, wt_rl, wd_ll, bd_ll, wd_rl, bd_rl)


def _sc_gather1(table, idx, eg):
    per_w = eg // NW
    mesh = plsc.VectorSubcoreMesh(core_axis_name="c", subcore_axis_name="s")

    @functools.partial(
        pl.kernel, mesh=mesh,
        out_type=jax.ShapeDtypeStruct((eg, TW), _f32),
        scratch_types=[
            pltpu.VMEM((per_w,), jnp.int32),
            pltpu.VMEM((CH, TW), _f32),
            pltpu.VMEM((CH, TW), _f32),
            pltpu.SemaphoreType.DMA,
            pltpu.SemaphoreType.DMA,
        ],
    )
    def gk(t_hbm, i_hbm, o_hbm, idxv, rows_a, rows_b, sem_a, sem_b):
        wid = lax.axis_index("s") * 2 + lax.axis_index("c")
        base = wid * per_w
        pltpu.sync_copy(i_hbm.at[pl.ds(base, per_w)], idxv)
        bufs = (rows_a, rows_b)
        sems = (sem_a, sem_b)
        pending = None
        for c in range(per_w // CH):
            buf, sem = bufs[c % 2], sems[c % 2]
            cp = pltpu.async_copy(t_hbm.at[idxv.at[pl.ds(c * CH, CH)]], buf, sem)
            if pending is not None:
                pcp, pbuf, pdst = pending
                pcp.wait()
                pltpu.sync_copy(pbuf, pdst)
            pending = (cp, buf, o_hbm.at[pl.ds(base + c * CH, CH)])
        pcp, pbuf, pdst = pending
        pcp.wait()
        pltpu.sync_copy(pbuf, pdst)

    return gk(table, idx)


def _sc_gather(t_ll, idx_ll, t_rl, idx_rl):
    return _sc_gather1(t_ll, idx_ll, EG_LL), _sc_gather1(t_rl, idx_rl, EG_RL)


def _edge_body(bk_k, g_ref, d_ref, t_ref, w2e_ref, b2e_ref, w2c_ref, b2c_ref,
               w3_ref, wde_ref, wdc_ref, hn_ref, xn_ref):
    b, k = bk_k
    g = g_ref[...]
    rep = lambda a: jnp.broadcast_to(a[:, None, :], (b, k, a.shape[1])).reshape(b * k, a.shape[1])
    xs = g[:, 128:132]
    xd = rep(t_ref[:, 128:132])
    xdiff = xs - xd
    d2 = jnp.sum(xdiff * xdiff, axis=1, keepdims=True)
    dij = jnp.sqrt(d2)
    u = xdiff / (dij + 1e-9)
    de = rep(d_ref[...])
    dijb = _r(dij)
    h1e = g[:, 0:64] + de[:, 0:64] + dijb * wde_ref[...]
    h1c = g[:, 64:128] + de[:, 64:128] + dijb * wdc_ref[...]
    me = _silu(jnp.dot(_silu(h1e), w2e_ref[...], preferred_element_type=_f32) + b2e_ref[...])
    mc = _silu(jnp.dot(_silu(h1c), w2c_ref[...], preferred_element_type=_f32) + b2c_ref[...])
    s = jnp.dot(mc, w3_ref[...], preferred_element_type=_f32)
    mx = s * u
    hn_ref[...] = jnp.sum(me.reshape(b, k, 64), axis=1)
    xn_ref[...] = jnp.sum(mx.reshape(b, k, 4), axis=1)


def _edge_agg(g, d, t_ll, w2e, b2e, w2c, b2c, w3, wde, wdc, k):
    be = 512
    bk = be * k
    full = lambda i: (0, 0)
    return pl.pallas_call(
        functools.partial(_edge_body, (be, k)),
        grid=(NPAD // be,),
        in_specs=[
            pl.BlockSpec((bk, TW), lambda i: (i, 0)),
            pl.BlockSpec((be, 128), lambda i: (i, 0)),
            pl.BlockSpec((be, TW), lambda i: (i, 0)),
            pl.BlockSpec((64, 64), full),
            pl.BlockSpec((1, 64), full),
            pl.BlockSpec((64, 64), full),
            pl.BlockSpec((1, 64), full),
            pl.BlockSpec((64, 1), full),
            pl.BlockSpec((1, 64), full),
            pl.BlockSpec((1, 64), full),
        ],
        out_specs=[
            pl.BlockSpec((be, 64), lambda i: (i, 0)),
            pl.BlockSpec((be, 4), lambda i: (i, 0)),
        ],
        out_shape=[
            jax.ShapeDtypeStruct((NPAD, 64), _f32),
            jax.ShapeDtypeStruct((NPAD, 4), _f32),
        ],
    )(g, d, t_ll, w2e, b2e, w2c, b2c, w3, wde, wdc)


def _node_body(h_ref, hnl_ref, hnr_ref, xnl_ref, xnr_ref, x_ref,
               wna_ref, wnb_ref, b1_ref, wn2_ref, b2_ref, ho_ref, xo_ref):
    h = h_ref[...]
    hn = hnl_ref[...] + hnr_ref[...]
    u = (jnp.dot(h, wna_ref[...], preferred_element_type=_f32)
         + jnp.dot(hn, wnb_ref[...], preferred_element_type=_f32) + b1_ref[...])
    ho_ref[...] = h + jnp.dot(_silu(u), wn2_ref[...], preferred_element_type=_f32) + b2_ref[...]
    xo_ref[...] = x_ref[...] + xnl_ref[...] + xnr_ref[...]


def _node_update(h, hn_ll, hn_rl, xn_ll, xn_rl, x, wna, wnb, b1, wn2, b2):
    bn = 1024
    full = lambda i: (0, 0)
    return pl.pallas_call(
        _node_body,
        grid=(NPAD // bn,),
        in_specs=[
            pl.BlockSpec((bn, 64), lambda i: (i, 0)),
            pl.BlockSpec((bn, 64), lambda i: (i, 0)),
            pl.BlockSpec((bn, 64), lambda i: (i, 0)),
            pl.BlockSpec((bn, 4), lambda i: (i, 0)),
            pl.BlockSpec((bn, 4), lambda i: (i, 0)),
            pl.BlockSpec((bn, 4), lambda i: (i, 0)),
            pl.BlockSpec((64, 64), full),
            pl.BlockSpec((64, 64), full),
            pl.BlockSpec((1, 64), full),
            pl.BlockSpec((64, 64), full),
            pl.BlockSpec((1, 64), full),
        ],
        out_specs=[
            pl.BlockSpec((bn, 64), lambda i: (i, 0)),
            pl.BlockSpec((bn, 4), lambda i: (i, 0)),
        ],
        out_shape=[
            jax.ShapeDtypeStruct((NPAD, 64), _f32),
            jax.ShapeDtypeStruct((NPAD, 4), _f32),
        ],
    )(h, hn_ll, hn_rl, xn_ll, xn_rl, x, wna, wnb, b1, wn2, b2)


def _dec_body(h_ref, x_ref, lp_ref, w1_ref, b1_ref, w2_ref, b2_ref,
              eh_ref, ex_ref):
    h = h_ref[...]
    a = _silu(jnp.dot(h, w1_ref[...], preferred_element_type=_f32) + b1_ref[...])
    eh_ref[...] = jnp.dot(a, w2_ref[...], preferred_element_type=_f32) + b2_ref[...]
    ex_ref[...] = x_ref[...] - lp_ref[...]


def _decode(h, x, lp, w1, b1, w2, b2):
    bn = 1024
    full = lambda i: (0, 0)
    return pl.pallas_call(
        _dec_body,
        grid=(NPAD // bn,),
        in_specs=[
            pl.BlockSpec((bn, 64), lambda i: (i, 0)),
            pl.BlockSpec((bn, 4), lambda i: (i, 0)),
            pl.BlockSpec((bn, 4), lambda i: (i, 0)),
            pl.BlockSpec((64, 64), full),
            pl.BlockSpec((1, 64), full),
            pl.BlockSpec((64, ATOM_NF), full),
            pl.BlockSpec((1, ATOM_NF), full),
        ],
        out_specs=[
            pl.BlockSpec((bn, ATOM_NF), lambda i: (i, 0)),
            pl.BlockSpec((bn, 4), lambda i: (i, 0)),
        ],
        out_shape=[
            jax.ShapeDtypeStruct((NPAD, ATOM_NF), _f32),
            jax.ShapeDtypeStruct((NPAD, 4), _f32),
        ],
    )(h, x, lp, w1, b1, w2, b2)


_gather_impl = _sc_gather


def kernel(lig_pos, lig_feat, rec_pos, rec_feat, timestep, params):
    lp = _pad_rows(lig_pos[0], NPAD)
    rp = _pad_rows(rec_pos[0], NPAD)
    lf = _pad_rows(lig_feat[0], NPAD)
    rf = _pad_rows(rec_feat[0], NPAD)
    t = timestep[0]

    xl = jnp.pad(lp, ((0, 0), (0, 1)))
    xr = jnp.pad(rp, ((0, 0), (0, 1)))

    def enc_w(ps):
        w2 = jnp.pad(ps[1]['W'], ((0, 0), (0, 1)))
        b2 = jnp.concatenate([ps[1]['b'], t.reshape(1)])
        return ps[0]['W'], ps[0]['b'].reshape(1, 64), w2, b2.reshape(1, 64)

    lw1, lb1, lw2, lb2 = enc_w(params['lig_enc'])
    rw1, rb1, rw2, rb2 = enc_w(params['rec_enc'])
    h2 = _encode_both(
        jnp.stack([lf, rf]),
        jnp.stack([lw1, rw1]), jnp.stack([lb1, rb1])[:, None, :].reshape(2, 1, 64),
        jnp.stack([lw2, rw2]), jnp.stack([lb2, rb2])[:, None, :].reshape(2, 1, 64),
    )
    h_lig, h_rec = h2[0], h2[1]

    def src_planes(pos):
        ssq = jnp.sum(pos[:, :3] * pos[:, :3], axis=1)
        ssq = jnp.where(jnp.arange(NPAD) < N, ssq, jnp.float32(jnp.inf))
        pl8 = jnp.zeros((8, NPAD), _f32)
        pl8 = pl8.at[0:3].set(pos[:, :3].T)
        return pl8.at[3].set(ssq)

    ll_idx = _knn(xl, src_planes(xl), LIG_K)
    rl_idx = _knn(xl, src_planes(xr), REC_K)
    idx_ll = ll_idx.reshape(-1)
    idx_rl = jnp.concatenate(
        [rl_idx.reshape(-1), jnp.arange(EG_RL - E_RL, dtype=jnp.int32) % N])

    x_lig = xl
    for layer in params['layers']:
        ew = layer['edge_ll'][0]['W']
        cw = layer['coord_ll'][0]['W']
        ew_r = layer['edge_rl'][0]['W']
        cw_r = layer['coord_rl'][0]['W']
        wt_ll = jnp.concatenate([ew[0:64], cw[0:64]], axis=1)
        wt_rl = jnp.concatenate([ew_r[0:64], cw_r[0:64]], axis=1)
        wd_ll = jnp.concatenate([ew[64:128], cw[64:128]], axis=1)
        wd_rl = jnp.concatenate([ew_r[64:128], cw_r[64:128]], axis=1)
        bd_ll = jnp.concatenate([layer['edge_ll'][0]['b'], layer['coord_ll'][0]['b']]).reshape(1, 128)
        bd_rl = jnp.concatenate([layer['edge_rl'][0]['b'], layer['coord_rl'][0]['b']]).reshape(1, 128)

        t_ll, t_rl, d_ll, d_rl = _project(
            h_lig, h_rec, x_lig, xr, wt_ll, wt_rl, wd_ll, bd_ll, wd_rl, bd_rl)

        g_ll, g_rl = _gather_impl(t_ll, idx_ll, t_rl, idx_rl)

        def etype_w(ps_e, ps_c):
            return (ps_e[1]['W'], ps_e[1]['b'].reshape(1, 64),
                    ps_c[1]['W'], ps_c[1]['b'].reshape(1, 64),
                    ps_c[2]['W'],
                    _r(ps_e[0]['W'][128:129]), _r(ps_c[0]['W'][128:129]))

        hn_ll, xn_ll = _edge_agg(g_ll, d_ll, t_ll,
                                 *etype_w(layer['edge_ll'], layer['coord_ll']), LIG_K)
        hn_rl, xn_rl = _edge_agg(g_rl, d_rl, t_ll,
                                 *etype_w(layer['edge_rl'], layer['coord_rl']), REC_K)

        nw = layer['node'][0]['W']
        h_lig, x_lig = _node_update(
            h_lig, hn_ll, hn_rl, xn_ll, xn_rl, x_lig,
            nw[0:64], nw[64:128], layer['node'][0]['b'].reshape(1, 64),
            layer['node'][1]['W'], layer['node'][1]['b'].reshape(1, 64))

    dw1 = jnp.pad(params['lig_dec'][0]['W'], ((0, 1), (0, 0)))
    eps_h, eps_x = _decode(
        h_lig, x_lig, xl, dw1, params['lig_dec'][0]['b'].reshape(1, 64),
        params['lig_dec'][1]['W'], params['lig_dec'][1]['b'].reshape(1, ATOM_NF))
    return eps_h[:N], eps_x[:N, :3]


# speedup vs baseline: 1.0847x; 1.0847x over previous
import functools

import jax
import jax.numpy as jnp
from jax import lax
from jax.experimental import pallas as pl
from jax.experimental.pallas import tpu as pltpu
from jax.experimental.pallas import tpu_sc as plsc

N = 10000
NPAD = 10240
ATOM_NF = 32
IN_SIZE = 64
LIG_K = 8
REC_K = 6
TW = 256
EG_LL = NPAD * LIG_K
E_RL = NPAD * REC_K
EG_RL = 65536
CH = 128
NW = 32
PER_W_LL = EG_LL // NW
PER_W_RL = EG_RL // NW

_f32 = jnp.float32
_BF = jnp.bfloat16


def _r(x):
    return x.astype(_BF).astype(jnp.float32)


def _silu(x):
    return x * jax.nn.sigmoid(x)


def _pad_rows(a, npad):
    return jnp.pad(a, ((0, npad - a.shape[0]),) + ((0, 0),) * (a.ndim - 1))


def _enc_body(x_ref, w1_ref, b1_ref, w2_ref, b2_ref, o_ref):
    x = x_ref[0]
    h = _silu(jnp.dot(x, w1_ref[0], preferred_element_type=_f32) + b1_ref[0])
    o_ref[0] = jnp.dot(h, w2_ref[0], preferred_element_type=_f32) + b2_ref[0]


def _encode_both(feat2, w1, b1, w2, b2):
    blk = 2048
    return pl.pallas_call(
        _enc_body,
        grid=(2, NPAD // blk),
        in_specs=[
            pl.BlockSpec((1, blk, ATOM_NF), lambda s, i: (s, i, 0)),
            pl.BlockSpec((1, ATOM_NF, 64), lambda s, i: (s, 0, 0)),
            pl.BlockSpec((1, 1, 64), lambda s, i: (s, 0, 0)),
            pl.BlockSpec((1, 64, 64), lambda s, i: (s, 0, 0)),
            pl.BlockSpec((1, 1, 64), lambda s, i: (s, 0, 0)),
        ],
        out_specs=pl.BlockSpec((1, blk, 64), lambda s, i: (s, i, 0)),
        out_shape=jax.ShapeDtypeStruct((2, NPAD, 64), _f32),
    )(feat2, w1, b1, w2, b2)


def _knn_body(k, dst_ref, srcp_ref, o_ref):
    d = dst_ref[...]
    s0 = srcp_ref[0:1, :]
    s1 = srcp_ref[1:2, :]
    s2 = srcp_ref[2:3, :]
    ssq = srcp_ref[4:5, :]
    dd = jnp.sum(d[:, 0:3] * d[:, 0:3], axis=1, keepdims=True)
    dot = jnp.dot(d, srcp_ref[0:4, :], preferred_element_type=_f32)
    acc = (dd + ssq) - 2.0 * dot
    iota = lax.broadcasted_iota(jnp.int32, acc.shape, 1)
    for j in range(k):
        m = jnp.min(acc, axis=1, keepdims=True)
        eq = acc == m
        idx = jnp.min(jnp.where(eq, iota, jnp.int32(2**30)), axis=1, keepdims=True)
        o_ref[:, j:j + 1] = idx
        acc = jnp.where(iota == idx, jnp.float32(jnp.inf), acc)


def _knn(dst_pos, srcp, k):
    rk = 256
    return pl.pallas_call(
        functools.partial(_knn_body, k),
        grid=(NPAD // rk,),
        in_specs=[
            pl.BlockSpec((rk, 4), lambda i: (i, 0)),
            pl.BlockSpec((8, NPAD), lambda i: (0, 0)),
        ],
        out_specs=pl.BlockSpec((rk, k), lambda i: (i, 0)),
        out_shape=jax.ShapeDtypeStruct((NPAD, k), jnp.int32),
    )(dst_pos, srcp)


def _proj_body(hl_ref, hr_ref, xl_ref, xr_ref, wt_ll_ref, wt_rl_ref,
               wd_ll_ref, bd_ll_ref, wd_rl_ref, bd_rl_ref,
               tll_ref, trl_ref, dll_ref, drl_ref):
    hl = hl_ref[...]
    hr = hr_ref[...]
    tll_ref[:, 0:128] = jnp.dot(hl, wt_ll_ref[...], preferred_element_type=_f32)
    tll_ref[:, 128:132] = xl_ref[...]
    tll_ref[:, 132:256] = jnp.zeros_like(tll_ref[:, 132:256])
    trl_ref[:, 0:128] = jnp.dot(hr, wt_rl_ref[...], preferred_element_type=_f32)
    trl_ref[:, 128:132] = xr_ref[...]
    trl_ref[:, 132:256] = jnp.zeros_like(trl_ref[:, 132:256])
    dll_ref[...] = jnp.dot(hl, wd_ll_ref[...], preferred_element_type=_f32) + bd_ll_ref[...]
    drl_ref[...] = jnp.dot(hl, wd_rl_ref[...], preferred_element_type=_f32) + bd_rl_ref[...]


def _project(h_lig, h_rec, x_lig, x_rec, wt_ll, wt_rl, wd_ll, bd_ll, wd_rl, bd_rl):
    bp = 1024
    full = lambda i: (0, 0)
    return pl.pallas_call(
        _proj_body,
        grid=(NPAD // bp,),
        in_specs=[
            pl.BlockSpec((bp, 64), lambda i: (i, 0)),
            pl.BlockSpec((bp, 64), lambda i: (i, 0)),
            pl.BlockSpec((bp, 4), lambda i: (i, 0)),
            pl.BlockSpec((bp, 4), lambda i: (i, 0)),
            pl.BlockSpec((64, 128), full),
            pl.BlockSpec((64, 128), full),
            pl.BlockSpec((64, 128), full),
            pl.BlockSpec((1, 128), full),
            pl.BlockSpec((64, 128), full),
            pl.BlockSpec((1, 128), full),
        ],
        out_specs=[
            pl.BlockSpec((bp, TW), lambda i: (i, 0)),
            pl.BlockSpec((bp, TW), lambda i: (i, 0)),
            pl.BlockSpec((bp, 128), lambda i: (i, 0)),
            pl.BlockSpec((bp, 128), lambda i: (i, 0)),
        ],
        out_shape=[
            jax.ShapeDtypeStruct((NPAD, TW), _f32),
            jax.ShapeDtypeStruct((NPAD, TW), _f32),
            jax.ShapeDtypeStruct((NPAD, 128), _f32),
            jax.ShapeDtypeStruct((NPAD, 128), _f32),
        ],
    )(h_lig, h_rec, x_lig, x_rec, wt_ll, wt_rl, wd_ll, bd_ll, wd_rl, bd_rl)


def _sc_gather1(table, idx, eg):
    per_w = eg // NW
    mesh = plsc.VectorSubcoreMesh(core_axis_name="c", subcore_axis_name="s")

    @functools.partial(
        pl.kernel, mesh=mesh,
        out_type=jax.ShapeDtypeStruct((eg, TW), _f32),
        scratch_types=[
            pltpu.VMEM((per_w,), jnp.int32),
            pltpu.VMEM((CH, TW), _f32),
            pltpu.VMEM((CH, TW), _f32),
            pltpu.SemaphoreType.DMA,
            pltpu.SemaphoreType.DMA,
        ],
    )
    def gk(t_hbm, i_hbm, o_hbm, idxv, rows_a, rows_b, sem_a, sem_b):
        wid = lax.axis_index("s") * 2 + lax.axis_index("c")
        base = wid * per_w
        pltpu.sync_copy(i_hbm.at[pl.ds(base, per_w)], idxv)
        bufs = (rows_a, rows_b)
        sems = (sem_a, sem_b)
        pending = None
        for c in range(per_w // CH):
            buf, sem = bufs[c % 2], sems[c % 2]
            cp = pltpu.async_copy(t_hbm.at[idxv.at[pl.ds(c * CH, CH)]], buf, sem)
            if pending is not None:
                pcp, pbuf, pdst = pending
                pcp.wait()
                pltpu.sync_copy(pbuf, pdst)
            pending = (cp, buf, o_hbm.at[pl.ds(base + c * CH, CH)])
        pcp, pbuf, pdst = pending
        pcp.wait()
        pltpu.sync_copy(pbuf, pdst)

    return gk(table, idx)


def _sc_gather(t_ll, idx_ll, t_rl, idx_rl):
    return _sc_gather1(t_ll, idx_ll, EG_LL), _sc_gather1(t_rl, idx_rl, EG_RL)


def _edge_body(bk_k, g_ref, d_ref, t_ref, w2e_ref, b2e_ref, w2c_ref, b2c_ref,
               w3_ref, wde_ref, wdc_ref, hn_ref, xn_ref):
    b, k = bk_k
    g = g_ref[...]
    rep = lambda a: jnp.broadcast_to(a[:, None, :], (b, k, a.shape[1])).reshape(b * k, a.shape[1])
    xs = g[:, 128:132]
    xd = rep(t_ref[:, 128:132])
    xdiff = xs - xd
    d2 = jnp.sum(xdiff * xdiff, axis=1, keepdims=True)
    dij = jnp.sqrt(d2)
    u = xdiff / (dij + 1e-9)
    de = rep(d_ref[...])
    dijb = _r(dij)
    h1e = g[:, 0:64] + de[:, 0:64] + dijb * wde_ref[...]
    h1c = g[:, 64:128] + de[:, 64:128] + dijb * wdc_ref[...]
    me = _silu(jnp.dot(_silu(h1e), w2e_ref[...], preferred_element_type=_f32) + b2e_ref[...])
    mc = _silu(jnp.dot(_silu(h1c), w2c_ref[...], preferred_element_type=_f32) + b2c_ref[...])
    s = jnp.dot(mc, w3_ref[...], preferred_element_type=_f32)
    mx = s * u
    hn_ref[...] = jnp.sum(me.reshape(b, k, 64), axis=1)
    xn_ref[...] = jnp.sum(mx.reshape(b, k, 4), axis=1)


def _edge_agg(g, d, t_ll, w2e, b2e, w2c, b2c, w3, wde, wdc, k):
    be = 512
    bk = be * k
    full = lambda i: (0, 0)
    return pl.pallas_call(
        functools.partial(_edge_body, (be, k)),
        grid=(NPAD // be,),
        in_specs=[
            pl.BlockSpec((bk, TW), lambda i: (i, 0)),
            pl.BlockSpec((be, 128), lambda i: (i, 0)),
            pl.BlockSpec((be, TW), lambda i: (i, 0)),
            pl.BlockSpec((64, 64), full),
            pl.BlockSpec((1, 64), full),
            pl.BlockSpec((64, 64), full),
            pl.BlockSpec((1, 64), full),
            pl.BlockSpec((64, 1), full),
            pl.BlockSpec((1, 64), full),
            pl.BlockSpec((1, 64), full),
        ],
        out_specs=[
            pl.BlockSpec((be, 64), lambda i: (i, 0)),
            pl.BlockSpec((be, 4), lambda i: (i, 0)),
        ],
        out_shape=[
            jax.ShapeDtypeStruct((NPAD, 64), _f32),
            jax.ShapeDtypeStruct((NPAD, 4), _f32),
        ],
    )(g, d, t_ll, w2e, b2e, w2c, b2c, w3, wde, wdc)


def _node_body(h_ref, hnl_ref, hnr_ref, xnl_ref, xnr_ref, x_ref,
               wna_ref, wnb_ref, b1_ref, wn2_ref, b2_ref, ho_ref, xo_ref):
    h = h_ref[...]
    hn = hnl_ref[...] + hnr_ref[...]
    u = (jnp.dot(h, wna_ref[...], preferred_element_type=_f32)
         + jnp.dot(hn, wnb_ref[...], preferred_element_type=_f32) + b1_ref[...])
    ho_ref[...] = h + jnp.dot(_silu(u), wn2_ref[...], preferred_element_type=_f32) + b2_ref[...]
    xo_ref[...] = x_ref[...] + xnl_ref[...] + xnr_ref[...]


def _node_update(h, hn_ll, hn_rl, xn_ll, xn_rl, x, wna, wnb, b1, wn2, b2):
    bn = 1024
    full = lambda i: (0, 0)
    return pl.pallas_call(
        _node_body,
        grid=(NPAD // bn,),
        in_specs=[
            pl.BlockSpec((bn, 64), lambda i: (i, 0)),
            pl.BlockSpec((bn, 64), lambda i: (i, 0)),
            pl.BlockSpec((bn, 64), lambda i: (i, 0)),
            pl.BlockSpec((bn, 4), lambda i: (i, 0)),
            pl.BlockSpec((bn, 4), lambda i: (i, 0)),
            pl.BlockSpec((bn, 4), lambda i: (i, 0)),
            pl.BlockSpec((64, 64), full),
            pl.BlockSpec((64, 64), full),
            pl.BlockSpec((1, 64), full),
            pl.BlockSpec((64, 64), full),
            pl.BlockSpec((1, 64), full),
        ],
        out_specs=[
            pl.BlockSpec((bn, 64), lambda i: (i, 0)),
            pl.BlockSpec((bn, 4), lambda i: (i, 0)),
        ],
        out_shape=[
            jax.ShapeDtypeStruct((NPAD, 64), _f32),
            jax.ShapeDtypeStruct((NPAD, 4), _f32),
        ],
    )(h, hn_ll, hn_rl, xn_ll, xn_rl, x, wna, wnb, b1, wn2, b2)


def _dec_body(h_ref, x_ref, lp_ref, w1_ref, b1_ref, w2_ref, b2_ref,
              eh_ref, ex_ref):
    h = h_ref[...]
    a = _silu(jnp.dot(h, w1_ref[...], preferred_element_type=_f32) + b1_ref[...])
    eh_ref[...] = jnp.dot(a, w2_ref[...], preferred_element_type=_f32) + b2_ref[...]
    ex_ref[...] = x_ref[...] - lp_ref[...]


def _decode(h, x, lp, w1, b1, w2, b2):
    bn = 1024
    full = lambda i: (0, 0)
    return pl.pallas_call(
        _dec_body,
        grid=(NPAD // bn,),
        in_specs=[
            pl.BlockSpec((bn, 64), lambda i: (i, 0)),
            pl.BlockSpec((bn, 4), lambda i: (i, 0)),
            pl.BlockSpec((bn, 4), lambda i: (i, 0)),
            pl.BlockSpec((64, 64), full),
            pl.BlockSpec((1, 64), full),
            pl.BlockSpec((64, ATOM_NF), full),
            pl.BlockSpec((1, ATOM_NF), full),
        ],
        out_specs=[
            pl.BlockSpec((bn, ATOM_NF), lambda i: (i, 0)),
            pl.BlockSpec((bn, 4), lambda i: (i, 0)),
        ],
        out_shape=[
            jax.ShapeDtypeStruct((NPAD, ATOM_NF), _f32),
            jax.ShapeDtypeStruct((NPAD, 4), _f32),
        ],
    )(h, x, lp, w1, b1, w2, b2)


_gather_impl = _sc_gather


def kernel(lig_pos, lig_feat, rec_pos, rec_feat, timestep, params):
    lp = _pad_rows(lig_pos[0], NPAD)
    rp = _pad_rows(rec_pos[0], NPAD)
    lf = _pad_rows(lig_feat[0], NPAD)
    rf = _pad_rows(rec_feat[0], NPAD)
    t = timestep[0]

    xl = jnp.pad(lp, ((0, 0), (0, 1)))
    xr = jnp.pad(rp, ((0, 0), (0, 1)))

    def enc_w(ps):
        w2 = jnp.pad(ps[1]['W'], ((0, 0), (0, 1)))
        b2 = jnp.concatenate([ps[1]['b'], t.reshape(1)])
        return ps[0]['W'], ps[0]['b'].reshape(1, 64), w2, b2.reshape(1, 64)

    lw1, lb1, lw2, lb2 = enc_w(params['lig_enc'])
    rw1, rb1, rw2, rb2 = enc_w(params['rec_enc'])
    h2 = _encode_both(
        jnp.stack([lf, rf]),
        jnp.stack([lw1, rw1]), jnp.stack([lb1, rb1])[:, None, :].reshape(2, 1, 64),
        jnp.stack([lw2, rw2]), jnp.stack([lb2, rb2])[:, None, :].reshape(2, 1, 64),
    )
    h_lig, h_rec = h2[0], h2[1]

    def src_planes(pos):
        ssq = jnp.sum(pos[:, :3] * pos[:, :3], axis=1)
        ssq = jnp.where(jnp.arange(NPAD) < N, ssq, jnp.float32(jnp.inf))
        pl8 = jnp.zeros((8, NPAD), _f32)
        pl8 = pl8.at[0:3].set(pos[:, :3].T)
        return pl8.at[4].set(ssq)

    ll_idx = _knn(xl, src_planes(xl), LIG_K)
    rl_idx = _knn(xl, src_planes(xr), REC_K)
    idx_ll = ll_idx.reshape(-1)
    idx_rl = jnp.concatenate(
        [rl_idx.reshape(-1), jnp.arange(EG_RL - E_RL, dtype=jnp.int32) % N])

    x_lig = xl
    for layer in params['layers']:
        ew = layer['edge_ll'][0]['W']
        cw = layer['coord_ll'][0]['W']
        ew_r = layer['edge_rl'][0]['W']
        cw_r = layer['coord_rl'][0]['W']
        wt_ll = jnp.concatenate([ew[0:64], cw[0:64]], axis=1)
        wt_rl = jnp.concatenate([ew_r[0:64], cw_r[0:64]], axis=1)
        wd_ll = jnp.concatenate([ew[64:128], cw[64:128]], axis=1)
        wd_rl = jnp.concatenate([ew_r[64:128], cw_r[64:128]], axis=1)
        bd_ll = jnp.concatenate([layer['edge_ll'][0]['b'], layer['coord_ll'][0]['b']]).reshape(1, 128)
        bd_rl = jnp.concatenate([layer['edge_rl'][0]['b'], layer['coord_rl'][0]['b']]).reshape(1, 128)

        t_ll, t_rl, d_ll, d_rl = _project(
            h_lig, h_rec, x_lig, xr, wt_ll, wt_rl, wd_ll, bd_ll, wd_rl, bd_rl)

        g_ll, g_rl = _gather_impl(t_ll, idx_ll, t_rl, idx_rl)

        def etype_w(ps_e, ps_c):
            return (ps_e[1]['W'], ps_e[1]['b'].reshape(1, 64),
                    ps_c[1]['W'], ps_c[1]['b'].reshape(1, 64),
                    ps_c[2]['W'],
                    _r(ps_e[0]['W'][128:129]), _r(ps_c[0]['W'][128:129]))

        hn_ll, xn_ll = _edge_agg(g_ll, d_ll, t_ll,
                                 *etype_w(layer['edge_ll'], layer['coord_ll']), LIG_K)
        hn_rl, xn_rl = _edge_agg(g_rl, d_rl, t_ll,
                                 *etype_w(layer['edge_rl'], layer['coord_rl']), REC_K)

        nw = layer['node'][0]['W']
        h_lig, x_lig = _node_update(
            h_lig, hn_ll, hn_rl, xn_ll, xn_rl, x_lig,
            nw[0:64], nw[64:128], layer['node'][0]['b'].reshape(1, 64),
            layer['node'][1]['W'], layer['node'][1]['b'].reshape(1, 64))

    dw1 = jnp.pad(params['lig_dec'][0]['W'], ((0, 1), (0, 0)))
    eps_h, eps_x = _decode(
        h_lig, x_lig, xl, dw1, params['lig_dec'][0]['b'].reshape(1, 64),
        params['lig_dec'][1]['W'], params['lig_dec'][1]['b'].reshape(1, ATOM_NF))
    return eps_h[:N], eps_x[:N, :3]
